# baseline (speedup 1.0000x reference)
import functools

import jax
import jax.numpy as jnp
from jax import lax
from jax.experimental import pallas as pl
from jax.experimental.pallas import tpu as pltpu
from jax.experimental.pallas import tpu_sc as plsc

EMB_SIZE = 64
SCALE = 8.0
CHUNK = 128


@functools.lru_cache(maxsize=None)
def _make_sc_kernel(B: int, D: int):
    info = plsc.get_sparse_core_info()
    nc, ns = info.num_cores, info.num_subcores
    nw = nc * ns
    b_per_w = B // nw
    n_chunks = b_per_w // CHUNK
    mesh = plsc.VectorSubcoreMesh(core_axis_name="c", subcore_axis_name="s")

    @functools.partial(
        pl.kernel,
        mesh=mesh,
        out_type=jax.ShapeDtypeStruct((B, D), jnp.float32),
        compiler_params=pltpu.CompilerParams(use_tc_tiling_on_sc=False),
        scratch_types=[
            pltpu.VMEM((n_chunks, CHUNK), jnp.int32),
            pltpu.VMEM((CHUNK, D), jnp.float32),
            pltpu.SemaphoreType.DMA,
        ],
    )
    def sc_embed(table_hbm, idx_hbm, out_hbm, idx_v, rows_v, sem):
        wid = lax.axis_index("s") * nc + lax.axis_index("c")
        pltpu.sync_copy(idx_hbm.at[wid], idx_v)

        def chunk_body(j, carry):
            pltpu.async_copy(table_hbm.at[idx_v.at[j]], rows_v, sem).wait()

            def scale_row(i, c):
                for t in range(D // 16):
                    sl = pl.ds(t * 16, 16)
                    rows_v[i, sl] = rows_v[i, sl] * SCALE
                return c

            lax.fori_loop(0, CHUNK, scale_row, 0)
            pltpu.sync_copy(
                rows_v, out_hbm.at[pl.ds(wid * b_per_w + j * CHUNK, CHUNK)]
            )
            return carry

        lax.fori_loop(0, n_chunks, chunk_body, 0)

    return sc_embed


@jax.jit
def kernel(tokens, emb_table):
    bsz, seq = tokens.shape
    B = bsz * seq
    info = plsc.get_sparse_core_info()
    nw = info.num_cores * info.num_subcores
    idx2d = tokens.reshape(-1).astype(jnp.int32).reshape(nw, B // (nw * CHUNK), CHUNK)
    out = _make_sc_kernel(B, EMB_SIZE)(emb_table, idx2d)
    return out.reshape(bsz, seq, EMB_SIZE)


# speedup vs baseline: 1.0806x; 1.0806x over previous
import functools

import jax
import jax.numpy as jnp
from jax import lax
from jax.experimental import pallas as pl
from jax.experimental.pallas import tpu as pltpu
from jax.experimental.pallas import tpu_sc as plsc

EMB_SIZE = 64
SCALE = 8.0
CHUNK = 128
NBUF = 5


@functools.lru_cache(maxsize=None)
def _make_sc_kernel(B: int, D: int):
    info = plsc.get_sparse_core_info()
    nc, ns = info.num_cores, info.num_subcores
    nw = nc * ns
    b_per_w = B // nw
    n_chunks = b_per_w // CHUNK
    n_outer = n_chunks // NBUF
    assert n_chunks % NBUF == 0
    mesh = plsc.VectorSubcoreMesh(core_axis_name="c", subcore_axis_name="s")

    @functools.partial(
        pl.kernel,
        mesh=mesh,
        out_type=jax.ShapeDtypeStruct((B, D), jnp.float32),
        compiler_params=pltpu.CompilerParams(use_tc_tiling_on_sc=False),
        scratch_types=(
            [pltpu.VMEM((n_chunks, CHUNK), jnp.int32)]
            + [pltpu.VMEM((CHUNK, D), jnp.float32) for _ in range(2 * NBUF)]
            + [pltpu.SemaphoreType.DMA for _ in range(2 * NBUF)]
        ),
    )
    def sc_embed(table_hbm, idx_hbm, out_hbm, idx_v, *bufs_and_sems):
        g_buf = bufs_and_sems[:NBUF]
        s_buf = bufs_and_sems[NBUF : 2 * NBUF]
        gsem = bufs_and_sems[2 * NBUF : 3 * NBUF]
        ssem = bufs_and_sems[3 * NBUF : 4 * NBUF]

        wid = lax.axis_index("s") * nc + lax.axis_index("c")
        out0 = wid * b_per_w
        pltpu.sync_copy(idx_hbm.at[wid], idx_v)

        def gather(j, b):
            return pltpu.make_async_copy(
                table_hbm.at[idx_v.at[j]], g_buf[b], gsem[b]
            )

        def scatter(j, b):
            return pltpu.make_async_copy(
                s_buf[b], out_hbm.at[pl.ds(out0 + j * CHUNK, CHUNK)], ssem[b]
            )

        for b in range(NBUF):
            gather(b, b).start()

        @pl.loop(0, n_outer)
        def outer(g):
            for b in range(NBUF):
                j = g * NBUF + b
                gather(j, b).wait()

                @pl.when(g > 0)
                def _():
                    scatter(j - NBUF, b).wait()

                @plsc.parallel_loop(0, CHUNK)
                def scale_row(i):
                    for t in range(D // 16):
                        sl = pl.ds(t * 16, 16)
                        s_buf[b][i, sl] = g_buf[b][i, sl] * SCALE

                @pl.when(g < n_outer - 1)
                def _():
                    gather(j + NBUF, b).start()

                scatter(j, b).start()

        for b in range(NBUF):
            scatter((n_outer - 1) * NBUF + b, b).wait()

    return sc_embed


@jax.jit
def kernel(tokens, emb_table):
    bsz, seq = tokens.shape
    B = bsz * seq
    info = plsc.get_sparse_core_info()
    nw = info.num_cores * info.num_subcores
    idx3d = tokens.reshape(-1).astype(jnp.int32).reshape(nw, B // (nw * CHUNK), CHUNK)
    out = _make_sc_kernel(B, EMB_SIZE)(emb_table, idx3d)
    return out.reshape(bsz, seq, EMB_SIZE)
